# Initial kernel scaffold; baseline (speedup 1.0000x reference)
import functools

import jax
import jax.numpy as jnp
from jax import lax
from jax.experimental import pallas as pl
from jax.experimental.pallas import tpu as pltpu
from jax.experimental.pallas import tpu_sc as plsc

C = 1.0
SK = 1.0
MIN_NORM = 1e-15


def _artanh(x):
    return jnp.arctanh(jnp.clip(x, -1.0 + 1e-7, 1.0 - 1e-7))


def _rownorm(x):
    return jnp.maximum(jnp.sqrt(jnp.sum(x * x, axis=-1, keepdims=True)), MIN_NORM)


def _ht_body(x_ref, w_ref, b_ref, o_ref):
    x = x_ref[...]
    w = w_ref[...]
    mx = lax.dot_general(x, w, (((1,), (1,)), ((), ())),
                         preferred_element_type=jnp.float32)
    xn = _rownorm(x)
    mxn = _rownorm(mx)
    res = jnp.tanh(mxn / xn * _artanh(SK * xn)) * mx / (mxn * SK)
    zero_row = jnp.all(mx == 0, axis=-1, keepdims=True)
    h = jnp.where(zero_row, jnp.zeros_like(res), res)
    bvec = b_ref[...]
    bn = _rownorm(bvec)
    b_h = jnp.tanh(SK * bn) * bvec / (SK * bn)
    k = -C
    x2 = jnp.sum(h * h, axis=-1, keepdims=True)
    y2 = jnp.sum(b_h * b_h, axis=-1, keepdims=True)
    xy = jnp.sum(h * b_h, axis=-1, keepdims=True)
    num = (1.0 - 2.0 * k * xy - k * y2) * h + (1.0 + k * x2) * b_h
    den = 1.0 - 2.0 * k * xy + (k * k) * x2 * y2
    h = num / jnp.maximum(den, MIN_NORM)
    hn = _rownorm(h)
    o_ref[...] = _artanh(SK * hn) * h / (SK * hn)


def _compute_ht(x_h, W, b):
    n, d = x_h.shape
    blk = 2000
    grid = n // blk
    return pl.pallas_call(
        _ht_body,
        grid=(grid,),
        in_specs=[
            pl.BlockSpec((blk, d), lambda i: (i, 0)),
            pl.BlockSpec((d, d), lambda i: (0, 0)),
            pl.BlockSpec((1, d), lambda i: (0, 0)),
        ],
        out_specs=pl.BlockSpec((blk, d), lambda i: (i, 0)),
        out_shape=jax.ShapeDtypeStruct((n, d), jnp.float32),
    )(x_h, W, b.reshape(1, d))


_NC = 2
_NS = 16
_NW = _NC * _NS


def _sc_agg(h_t, src_r, dst_r, zeros_nd):
    n, d = h_t.shape
    nch, chunk = src_r.shape[1], src_r.shape[2]
    rows_per_tile = n // _NS

    mesh = plsc.VectorSubcoreMesh(core_axis_name="c", subcore_axis_name="s")

    @functools.partial(
        pl.kernel,
        mesh=mesh,
        out_type=jax.ShapeDtypeStruct((_NC, n, d), jnp.float32),
        scratch_types=[
            pltpu.VMEM((nch, chunk), jnp.int32),
            pltpu.VMEM((nch, chunk), jnp.int32),
            pltpu.VMEM((chunk, d), jnp.float32),
            pltpu.VMEM_SHARED((n, d), jnp.float32),
            pltpu.SemaphoreType.DMA,
        ],
    )
    def agg_kernel(ht_hbm, srcr_hbm, dstr_hbm, zeros_hbm, out_hbm,
                   sidx_v, didx_v, rows_v, acc_sh, gsem):
        cid = lax.axis_index("c")
        sid = lax.axis_index("s")
        wid = sid * _NC + cid
        row0 = sid * rows_per_tile
        pltpu.sync_copy(zeros_hbm.at[pl.ds(row0, rows_per_tile)],
                        acc_sh.at[pl.ds(row0, rows_per_tile)])
        pltpu.sync_copy(srcr_hbm.at[wid], sidx_v)
        pltpu.sync_copy(dstr_hbm.at[wid], didx_v)
        plsc.subcore_barrier()

        def body(j, carry):
            pltpu.async_copy(ht_hbm.at[sidx_v.at[j]], rows_v, gsem).wait()
            pltpu.sync_copy(rows_v, acc_sh.at[didx_v.at[j]], add=True)
            return carry

        lax.fori_loop(0, nch, body, 0)
        plsc.subcore_barrier()
        pltpu.sync_copy(acc_sh.at[pl.ds(row0, rows_per_tile)],
                        out_hbm.at[cid].at[pl.ds(row0, rows_per_tile)])

    return agg_kernel(h_t, src_r, dst_r, zeros_nd)


def _post_body(p_ref, ht_ref, a_ref, o_ref):
    agg = p_ref[0] + p_ref[1] + ht_ref[...]
    an = _rownorm(agg)
    h2 = jnp.tanh(SK * an) * agg / (SK * an)
    h2n = _rownorm(h2)
    h2t = _artanh(SK * h2n) * h2 / (SK * h2n)
    a = a_ref[0]
    h2t = jnp.where(h2t >= 0, h2t, a * h2t)
    tn = _rownorm(h2t)
    h2 = jnp.tanh(SK * tn) * h2t / (SK * tn)
    eps = 4e-3
    maxnorm = (1.0 - eps) / SK
    nrm = _rownorm(h2)
    o_ref[...] = jnp.where(nrm > maxnorm, h2 / nrm * maxnorm, h2)


def _post(partials, h_t, a):
    n, d = h_t.shape
    blk = 2000
    grid = n // blk
    return pl.pallas_call(
        _post_body,
        grid=(grid,),
        in_specs=[
            pl.BlockSpec((_NC, blk, d), lambda i: (0, i, 0)),
            pl.BlockSpec((blk, d), lambda i: (i, 0)),
            pl.BlockSpec(memory_space=pltpu.SMEM),
        ],
        out_specs=pl.BlockSpec((blk, d), lambda i: (i, 0)),
        out_shape=jax.ShapeDtypeStruct((n, d), jnp.float32),
    )(partials, h_t, a)


def kernel(x_h, edge_index, W, b, a):
    n, d = x_h.shape
    e = edge_index.shape[1]
    per_w = e // _NW
    chunk = 80
    nch = per_w // chunk
    assert per_w * _NW == e and nch * chunk == per_w

    h_t = _compute_ht(x_h, W, b)
    src_r = edge_index[0].reshape(_NW, nch, chunk)
    dst_r = edge_index[1].reshape(_NW, nch, chunk)
    zeros_nd = jnp.zeros((n, d), jnp.float32)
    partials = _sc_agg(h_t, src_r, dst_r, zeros_nd)
    return _post(partials, h_t, a)


# speedup vs baseline: 6.7044x; 6.7044x over previous
import functools

import jax
import jax.numpy as jnp
from jax import lax
from jax.experimental import pallas as pl
from jax.experimental.pallas import tpu as pltpu
from jax.experimental.pallas import tpu_sc as plsc

C = 1.0
SK = 1.0
MIN_NORM = 1e-15


def _artanh(x):
    xc = jnp.clip(x, -1.0 + 1e-7, 1.0 - 1e-7)
    return 0.5 * jnp.log((1.0 + xc) / (1.0 - xc))


def _rownorm(x):
    return jnp.maximum(jnp.sqrt(jnp.sum(x * x, axis=-1, keepdims=True)), MIN_NORM)


def _ht_body(x_ref, w_ref, b_ref, o_ref):
    x = x_ref[...]
    w = w_ref[...]
    mx = lax.dot_general(x, w, (((1,), (1,)), ((), ())),
                         preferred_element_type=jnp.float32)
    xn = _rownorm(x)
    mxn = _rownorm(mx)
    res = jnp.tanh(mxn / xn * _artanh(SK * xn)) * mx / (mxn * SK)
    zero_row = jnp.all(mx == 0, axis=-1, keepdims=True)
    h = jnp.where(zero_row, jnp.zeros_like(res), res)
    bvec = b_ref[...]
    bn = _rownorm(bvec)
    b_h = jnp.tanh(SK * bn) * bvec / (SK * bn)
    k = -C
    x2 = jnp.sum(h * h, axis=-1, keepdims=True)
    y2 = jnp.sum(b_h * b_h, axis=-1, keepdims=True)
    xy = jnp.sum(h * b_h, axis=-1, keepdims=True)
    num = (1.0 - 2.0 * k * xy - k * y2) * h + (1.0 + k * x2) * b_h
    den = 1.0 - 2.0 * k * xy + (k * k) * x2 * y2
    h = num / jnp.maximum(den, MIN_NORM)
    hn = _rownorm(h)
    o_ref[...] = _artanh(SK * hn) * h / (SK * hn)


def _compute_ht(x_h, W, b):
    n, d = x_h.shape
    blk = 2000
    grid = n // blk
    return pl.pallas_call(
        _ht_body,
        grid=(grid,),
        in_specs=[
            pl.BlockSpec((blk, d), lambda i: (i, 0)),
            pl.BlockSpec((d, d), lambda i: (0, 0)),
            pl.BlockSpec((1, d), lambda i: (0, 0)),
        ],
        out_specs=pl.BlockSpec((blk, d), lambda i: (i, 0)),
        out_shape=jax.ShapeDtypeStruct((n, d), jnp.float32),
    )(x_h, W, b.reshape(1, d))


_NC = 2
_NS = 16
_NW = _NC * _NS


def _sc_agg(h_t, src_r, dst_r, zeros_nd):
    n, d = h_t.shape
    nch, chunk = src_r.shape[1], src_r.shape[2]
    rpt = (n // _NS) & ~7
    tail0 = _NS * rpt
    tail = n - tail0

    mesh = plsc.VectorSubcoreMesh(core_axis_name="c", subcore_axis_name="s")

    @functools.partial(
        pl.kernel,
        mesh=mesh,
        out_type=jax.ShapeDtypeStruct((_NC, n, d), jnp.float32),
        scratch_types=[
            pltpu.VMEM((nch, chunk), jnp.int32),
            pltpu.VMEM((nch, chunk), jnp.int32),
            pltpu.VMEM((chunk, d), jnp.float32),
            pltpu.VMEM_SHARED((n, d), jnp.float32),
            pltpu.SemaphoreType.DMA,
        ],
    )
    def agg_kernel(ht_hbm, srcr_hbm, dstr_hbm, zeros_hbm, out_hbm,
                   sidx_v, didx_v, rows_v, acc_sh, gsem):
        cid = lax.axis_index("c")
        sid = lax.axis_index("s")
        wid = sid * _NC + cid
        row0 = sid * rpt
        pltpu.sync_copy(zeros_hbm.at[pl.ds(row0, rpt)],
                        acc_sh.at[pl.ds(row0, rpt)])
        if tail:
            @pl.when(sid == _NS - 1)
            def _():
                pltpu.sync_copy(zeros_hbm.at[pl.ds(tail0, tail)],
                                acc_sh.at[pl.ds(tail0, tail)])
        pltpu.sync_copy(srcr_hbm.at[wid], sidx_v)
        pltpu.sync_copy(dstr_hbm.at[wid], didx_v)
        plsc.subcore_barrier()

        def body(j, carry):
            pltpu.async_copy(ht_hbm.at[sidx_v.at[j]], rows_v, gsem).wait()
            pltpu.sync_copy(rows_v, acc_sh.at[didx_v.at[j]], add=True)
            return carry

        lax.fori_loop(0, nch, body, 0)
        plsc.subcore_barrier()
        pltpu.sync_copy(acc_sh.at[pl.ds(row0, rpt)],
                        out_hbm.at[cid].at[pl.ds(row0, rpt)])
        if tail:
            @pl.when(sid == _NS - 1)
            def _():
                pltpu.sync_copy(acc_sh.at[pl.ds(tail0, tail)],
                                out_hbm.at[cid].at[pl.ds(tail0, tail)])

    return agg_kernel(h_t, src_r, dst_r, zeros_nd)


def _post_body(p_ref, ht_ref, a_ref, o_ref):
    agg = p_ref[0] + p_ref[1] + ht_ref[...]
    an = _rownorm(agg)
    h2 = jnp.tanh(SK * an) * agg / (SK * an)
    h2n = _rownorm(h2)
    h2t = _artanh(SK * h2n) * h2 / (SK * h2n)
    a = a_ref[0]
    h2t = jnp.where(h2t >= 0, h2t, a * h2t)
    tn = _rownorm(h2t)
    h2 = jnp.tanh(SK * tn) * h2t / (SK * tn)
    eps = 4e-3
    maxnorm = (1.0 - eps) / SK
    nrm = _rownorm(h2)
    o_ref[...] = jnp.where(nrm > maxnorm, h2 / nrm * maxnorm, h2)


def _post(partials, h_t, a):
    n, d = h_t.shape
    blk = 2000
    grid = n // blk
    return pl.pallas_call(
        _post_body,
        grid=(grid,),
        in_specs=[
            pl.BlockSpec((_NC, blk, d), lambda i: (0, i, 0)),
            pl.BlockSpec((blk, d), lambda i: (i, 0)),
            pl.BlockSpec(memory_space=pltpu.SMEM),
        ],
        out_specs=pl.BlockSpec((blk, d), lambda i: (i, 0)),
        out_shape=jax.ShapeDtypeStruct((n, d), jnp.float32),
    )(partials, h_t, a)


def kernel(x_h, edge_index, W, b, a):
    n, d = x_h.shape
    e = edge_index.shape[1]
    per_w = e // _NW
    chunk = 80
    nch = per_w // chunk
    assert per_w * _NW == e and nch * chunk == per_w

    h_t = _compute_ht(x_h, W, b)
    src_r = edge_index[0].reshape(_NW, nch, chunk)
    dst_r = edge_index[1].reshape(_NW, nch, chunk)
    zeros_nd = jnp.zeros((n, d), jnp.float32)
    partials = _sc_agg(h_t, src_r, dst_r, zeros_nd)
    return _post(partials, h_t, a)


# speedup vs baseline: 10.2016x; 1.5216x over previous
import functools

import jax
import jax.numpy as jnp
from jax import lax
from jax.experimental import pallas as pl
from jax.experimental.pallas import tpu as pltpu
from jax.experimental.pallas import tpu_sc as plsc

C = 1.0
SK = 1.0
MIN_NORM = 1e-15


def _artanh(x):
    xc = jnp.clip(x, -1.0 + 1e-7, 1.0 - 1e-7)
    return 0.5 * jnp.log((1.0 + xc) / (1.0 - xc))


def _rownorm(x):
    return jnp.maximum(jnp.sqrt(jnp.sum(x * x, axis=-1, keepdims=True)), MIN_NORM)


def _ht_body(x_ref, w_ref, b_ref, o_ref):
    x = x_ref[...]
    w = w_ref[...]
    mx = lax.dot_general(x, w, (((1,), (1,)), ((), ())),
                         preferred_element_type=jnp.float32)
    xn = _rownorm(x)
    mxn = _rownorm(mx)
    res = jnp.tanh(mxn / xn * _artanh(SK * xn)) * mx / (mxn * SK)
    zero_row = jnp.all(mx == 0, axis=-1, keepdims=True)
    h = jnp.where(zero_row, jnp.zeros_like(res), res)
    bvec = b_ref[...]
    bn = _rownorm(bvec)
    b_h = jnp.tanh(SK * bn) * bvec / (SK * bn)
    k = -C
    x2 = jnp.sum(h * h, axis=-1, keepdims=True)
    y2 = jnp.sum(b_h * b_h, axis=-1, keepdims=True)
    xy = jnp.sum(h * b_h, axis=-1, keepdims=True)
    num = (1.0 - 2.0 * k * xy - k * y2) * h + (1.0 + k * x2) * b_h
    den = 1.0 - 2.0 * k * xy + (k * k) * x2 * y2
    h = num / jnp.maximum(den, MIN_NORM)
    hn = _rownorm(h)
    o_ref[...] = _artanh(SK * hn) * h / (SK * hn)


def _compute_ht(x_h, W, b):
    n, d = x_h.shape
    blk = 2000
    grid = n // blk
    return pl.pallas_call(
        _ht_body,
        grid=(grid,),
        in_specs=[
            pl.BlockSpec((blk, d), lambda i: (i, 0)),
            pl.BlockSpec((d, d), lambda i: (0, 0)),
            pl.BlockSpec((1, d), lambda i: (0, 0)),
        ],
        out_specs=pl.BlockSpec((blk, d), lambda i: (i, 0)),
        out_shape=jax.ShapeDtypeStruct((n, d), jnp.float32),
    )(x_h, W, b.reshape(1, d))


_NC = 2
_NS = 16
_NW = _NC * _NS


def _sc_agg(h_t, pidx, zeros_nd):
    n, d = h_t.shape
    nch, chunk = pidx.shape[1], pidx.shape[3]
    assert nch % 4 == 0 and nch >= 8
    rpt = (n // _NS) & ~7
    tail0 = _NS * rpt
    tail = n - tail0

    mesh = plsc.VectorSubcoreMesh(core_axis_name="c", subcore_axis_name="s")

    @functools.partial(
        pl.kernel,
        mesh=mesh,
        out_type=jax.ShapeDtypeStruct((_NC, n, d), jnp.float32),
        scratch_types=[
            pltpu.VMEM((4, 2, chunk), jnp.int32),
            pltpu.VMEM((2, chunk, d), jnp.float32),
            pltpu.VMEM_SHARED((n, d), jnp.float32),
            pltpu.SemaphoreType.DMA,
            pltpu.SemaphoreType.DMA,
            pltpu.SemaphoreType.DMA,
            pltpu.SemaphoreType.DMA,
        ],
    )
    def agg_kernel(ht_hbm, pidx_hbm, zeros_hbm, out_hbm,
                   ring_v, rows2_v, acc_sh, sem_i0, sem_i1, sem_g0, sem_g1):
        sem_i = (sem_i0, sem_i1)
        sem_g = (sem_g0, sem_g1)
        cid = lax.axis_index("c")
        sid = lax.axis_index("s")
        wid = sid * _NC + cid
        row0 = sid * rpt
        pltpu.sync_copy(zeros_hbm.at[pl.ds(row0, rpt)],
                        acc_sh.at[pl.ds(row0, rpt)])
        if tail:
            @pl.when(sid == _NS - 1)
            def _():
                pltpu.sync_copy(zeros_hbm.at[pl.ds(tail0, tail)],
                                acc_sh.at[pl.ds(tail0, tail)])
        plsc.subcore_barrier()

        my_idx = pidx_hbm.at[wid]

        def idx_start(j, s4, p2):
            pltpu.async_copy(my_idx.at[j], ring_v.at[s4], sem_i[p2])

        def idx_wait(j, s4, p2):
            pltpu.make_async_copy(my_idx.at[j], ring_v.at[s4], sem_i[p2]).wait()

        def gat_start(g, s4, p2):
            pltpu.async_copy(ht_hbm.at[ring_v.at[s4].at[0]], rows2_v.at[p2],
                             sem_g[p2])

        def gat_wait(g, s4, p2):
            pltpu.make_async_copy(ht_hbm.at[ring_v.at[s4].at[0]],
                                  rows2_v.at[p2], sem_g[p2]).wait()

        def scat(g, s4, p2):
            pltpu.sync_copy(rows2_v.at[p2], acc_sh.at[ring_v.at[s4].at[1]],
                            add=True)

        idx_start(0, 0, 0)
        idx_start(1, 1, 1)
        idx_wait(0, 0, 0)
        gat_start(0, 0, 0)
        idx_start(2, 2, 0)
        idx_wait(1, 1, 1)
        gat_start(1, 1, 1)
        idx_start(3, 3, 1)
        gat_wait(0, 0, 0)
        scat(0, 0, 0)

        def body(t, carry):
            g0 = 4 * t + 2
            for m in range(4):
                g = g0 + m
                s4 = (2 + m) % 4
                p2 = m % 2
                idx_wait(g, s4, p2)
                gat_start(g, s4, p2)
                idx_start(g + 2, m % 4, p2)
                gat_wait(g - 1, (1 + m) % 4, 1 - p2)
                scat(g - 1, (1 + m) % 4, 1 - p2)
            return carry

        lax.fori_loop(0, (nch - 4) // 4, body, 0)

        idx_wait(nch - 2, 2, 0)
        gat_start(nch - 2, 2, 0)
        gat_wait(nch - 3, 1, 1)
        scat(nch - 3, 1, 1)
        idx_wait(nch - 1, 3, 1)
        gat_start(nch - 1, 3, 1)
        gat_wait(nch - 2, 2, 0)
        scat(nch - 2, 2, 0)
        gat_wait(nch - 1, 3, 1)
        scat(nch - 1, 3, 1)

        plsc.subcore_barrier()
        pltpu.sync_copy(acc_sh.at[pl.ds(row0, rpt)],
                        out_hbm.at[cid].at[pl.ds(row0, rpt)])
        if tail:
            @pl.when(sid == _NS - 1)
            def _():
                pltpu.sync_copy(acc_sh.at[pl.ds(tail0, tail)],
                                out_hbm.at[cid].at[pl.ds(tail0, tail)])

    return agg_kernel(h_t, pidx, zeros_nd)


def _post_body(p_ref, ht_ref, a_ref, o_ref):
    agg = p_ref[0] + p_ref[1] + ht_ref[...]
    an = _rownorm(agg)
    h2 = jnp.tanh(SK * an) * agg / (SK * an)
    h2n = _rownorm(h2)
    h2t = _artanh(SK * h2n) * h2 / (SK * h2n)
    a = a_ref[0]
    h2t = jnp.where(h2t >= 0, h2t, a * h2t)
    tn = _rownorm(h2t)
    h2 = jnp.tanh(SK * tn) * h2t / (SK * tn)
    eps = 4e-3
    maxnorm = (1.0 - eps) / SK
    nrm = _rownorm(h2)
    o_ref[...] = jnp.where(nrm > maxnorm, h2 / nrm * maxnorm, h2)


def _post(partials, h_t, a):
    n, d = h_t.shape
    blk = 2000
    grid = n // blk
    return pl.pallas_call(
        _post_body,
        grid=(grid,),
        in_specs=[
            pl.BlockSpec((_NC, blk, d), lambda i: (0, i, 0)),
            pl.BlockSpec((blk, d), lambda i: (i, 0)),
            pl.BlockSpec(memory_space=pltpu.SMEM),
        ],
        out_specs=pl.BlockSpec((blk, d), lambda i: (i, 0)),
        out_shape=jax.ShapeDtypeStruct((n, d), jnp.float32),
    )(partials, h_t, a)


def kernel(x_h, edge_index, W, b, a):
    n, d = x_h.shape
    e = edge_index.shape[1]
    per_w = e // _NW
    chunk = 125
    nch = per_w // chunk
    assert per_w * _NW == e and nch * chunk == per_w

    h_t = _compute_ht(x_h, W, b)
    src_r = edge_index[0].reshape(_NW, nch, chunk)
    dst_r = edge_index[1].reshape(_NW, nch, chunk)
    pidx = jnp.stack([src_r, dst_r], axis=2)
    zeros_nd = jnp.zeros((n, d), jnp.float32)
    partials = _sc_agg(h_t, pidx, zeros_nd)
    return _post(partials, h_t, a)


# speedup vs baseline: 10.4202x; 1.0214x over previous
import functools

import jax
import jax.numpy as jnp
from jax import lax
from jax.experimental import pallas as pl
from jax.experimental.pallas import tpu as pltpu
from jax.experimental.pallas import tpu_sc as plsc

C = 1.0
SK = 1.0
MIN_NORM = 1e-15


def _artanh(x):
    xc = jnp.clip(x, -1.0 + 1e-7, 1.0 - 1e-7)
    return 0.5 * jnp.log((1.0 + xc) / (1.0 - xc))


def _rownorm(x):
    return jnp.maximum(jnp.sqrt(jnp.sum(x * x, axis=-1, keepdims=True)), MIN_NORM)


def _ht_body(x_ref, w_ref, b_ref, o_ref):
    x = x_ref[...]
    w = w_ref[...]
    mx = lax.dot_general(x, w, (((1,), (1,)), ((), ())),
                         preferred_element_type=jnp.float32)
    xn = _rownorm(x)
    mxn = _rownorm(mx)
    res = jnp.tanh(mxn / xn * _artanh(SK * xn)) * mx / (mxn * SK)
    zero_row = jnp.all(mx == 0, axis=-1, keepdims=True)
    h = jnp.where(zero_row, jnp.zeros_like(res), res)
    bvec = b_ref[...]
    bn = _rownorm(bvec)
    b_h = jnp.tanh(SK * bn) * bvec / (SK * bn)
    k = -C
    x2 = jnp.sum(h * h, axis=-1, keepdims=True)
    y2 = jnp.sum(b_h * b_h, axis=-1, keepdims=True)
    xy = jnp.sum(h * b_h, axis=-1, keepdims=True)
    num = (1.0 - 2.0 * k * xy - k * y2) * h + (1.0 + k * x2) * b_h
    den = 1.0 - 2.0 * k * xy + (k * k) * x2 * y2
    h = num / jnp.maximum(den, MIN_NORM)
    hn = _rownorm(h)
    o_ref[...] = _artanh(SK * hn) * h / (SK * hn)


def _compute_ht(x_h, W, b):
    n, d = x_h.shape
    blk = 2000
    grid = n // blk
    return pl.pallas_call(
        _ht_body,
        grid=(grid,),
        in_specs=[
            pl.BlockSpec((blk, d), lambda i: (i, 0)),
            pl.BlockSpec((d, d), lambda i: (0, 0)),
            pl.BlockSpec((1, d), lambda i: (0, 0)),
        ],
        out_specs=pl.BlockSpec((blk, d), lambda i: (i, 0)),
        out_shape=jax.ShapeDtypeStruct((n, d), jnp.float32),
    )(x_h, W, b.reshape(1, d))


_NC = 2
_NS = 16
_NW = _NC * _NS


def _sc_agg(h_t, pidx):
    n, d = h_t.shape
    nch, chunk = pidx.shape[1] // 2, pidx.shape[2]
    nq = nch // 4
    assert nch % 8 == 0 and nq >= 3
    rpt = (n // _NS) & ~7
    tail0 = _NS * rpt
    tail = n - tail0

    mesh = plsc.VectorSubcoreMesh(core_axis_name="c", subcore_axis_name="s")

    @functools.partial(
        pl.kernel,
        mesh=mesh,
        out_type=jax.ShapeDtypeStruct((_NC, n, d), jnp.float32),
        scratch_types=[
            pltpu.VMEM((2, 8, chunk), jnp.int32),
            pltpu.VMEM((2, chunk, d), jnp.float32),
            pltpu.VMEM_SHARED((n, d), jnp.float32),
            pltpu.SemaphoreType.DMA,
            pltpu.SemaphoreType.DMA,
            pltpu.SemaphoreType.DMA,
        ],
    )
    def agg_kernel(ht_hbm, pidx_hbm, out_hbm,
                   ring_v, rows2_v, acc_sh, sem_i, sem_g0, sem_g1):
        sem_g = (sem_g0, sem_g1)
        cid = lax.axis_index("c")
        sid = lax.axis_index("s")
        wid = sid * _NC + cid
        row0 = sid * rpt
        pltpu.sync_copy(ht_hbm.at[pl.ds(row0, rpt)],
                        acc_sh.at[pl.ds(row0, rpt)])
        if tail:
            @pl.when(sid == _NS - 1)
            def _():
                pltpu.sync_copy(ht_hbm.at[pl.ds(tail0, tail)],
                                acc_sh.at[pl.ds(tail0, tail)])
        plsc.subcore_barrier()

        my_idx = pidx_hbm.at[wid]

        def idx_start(q, s2):
            pltpu.async_copy(my_idx.at[pl.ds(8 * q, 8)], ring_v.at[s2], sem_i)

        def idx_wait(q, s2):
            pltpu.make_async_copy(my_idx.at[pl.ds(8 * q, 8)], ring_v.at[s2],
                                  sem_i).wait()

        def gat_start(g, s2, m, p2):
            pltpu.async_copy(ht_hbm.at[ring_v.at[s2].at[2 * m]],
                             rows2_v.at[p2], sem_g[p2])

        def gat_wait(g, s2, m, p2):
            pltpu.make_async_copy(ht_hbm.at[ring_v.at[s2].at[2 * m]],
                                  rows2_v.at[p2], sem_g[p2]).wait()

        def scat(g, s2, m, p2):
            pltpu.sync_copy(rows2_v.at[p2], acc_sh.at[ring_v.at[s2].at[2 * m + 1]],
                            add=True)

        idx_start(0, 0)
        idx_wait(0, 0)
        gat_start(0, 0, 0, 0)
        idx_start(1, 1)
        for m in (1, 2, 3):
            gat_start(m, 0, m, m % 2)
            gat_wait(m - 1, 0, m - 1, (m - 1) % 2)
            scat(m - 1, 0, m - 1, (m - 1) % 2)

        def body(t, carry):
            for h in range(2):
                q = 2 * t + 1 + h
                s2 = (1 + h) % 2
                os2 = h % 2
                idx_wait(q, s2)
                gat_start(4 * q, s2, 0, 0)
                gat_wait(4 * q - 1, os2, 3, 1)
                scat(4 * q - 1, os2, 3, 1)
                idx_start(q + 1, os2)
                for m in (1, 2, 3):
                    gat_start(4 * q + m, s2, m, m % 2)
                    gat_wait(4 * q + m - 1, s2, m - 1, (m - 1) % 2)
                    scat(4 * q + m - 1, s2, m - 1, (m - 1) % 2)
            return carry

        lax.fori_loop(0, (nq - 2) // 2, body, 0)

        q = nq - 1
        idx_wait(q, 1)
        gat_start(4 * q, 1, 0, 0)
        gat_wait(4 * q - 1, 0, 3, 1)
        scat(4 * q - 1, 0, 3, 1)
        for m in (1, 2, 3):
            gat_start(4 * q + m, 1, m, m % 2)
            gat_wait(4 * q + m - 1, 1, m - 1, (m - 1) % 2)
            scat(4 * q + m - 1, 1, m - 1, (m - 1) % 2)
        gat_wait(nch - 1, 1, 3, 1)
        scat(nch - 1, 1, 3, 1)

        plsc.subcore_barrier()
        pltpu.sync_copy(acc_sh.at[pl.ds(row0, rpt)],
                        out_hbm.at[cid].at[pl.ds(row0, rpt)])
        if tail:
            @pl.when(sid == _NS - 1)
            def _():
                pltpu.sync_copy(acc_sh.at[pl.ds(tail0, tail)],
                                out_hbm.at[cid].at[pl.ds(tail0, tail)])

    return agg_kernel(h_t, pidx)


def _post_body(p_ref, ht_ref, a_ref, o_ref):
    agg = p_ref[0] + p_ref[1] - ht_ref[...]
    an = _rownorm(agg)
    h2 = jnp.tanh(SK * an) * agg / (SK * an)
    h2n = _rownorm(h2)
    h2t = _artanh(SK * h2n) * h2 / (SK * h2n)
    a = a_ref[0]
    h2t = jnp.where(h2t >= 0, h2t, a * h2t)
    tn = _rownorm(h2t)
    h2 = jnp.tanh(SK * tn) * h2t / (SK * tn)
    eps = 4e-3
    maxnorm = (1.0 - eps) / SK
    nrm = _rownorm(h2)
    o_ref[...] = jnp.where(nrm > maxnorm, h2 / nrm * maxnorm, h2)


def _post(partials, h_t, a):
    n, d = h_t.shape
    blk = 2000
    grid = n // blk
    return pl.pallas_call(
        _post_body,
        grid=(grid,),
        in_specs=[
            pl.BlockSpec((_NC, blk, d), lambda i: (0, i, 0)),
            pl.BlockSpec((blk, d), lambda i: (i, 0)),
            pl.BlockSpec(memory_space=pltpu.SMEM),
        ],
        out_specs=pl.BlockSpec((blk, d), lambda i: (i, 0)),
        out_shape=jax.ShapeDtypeStruct((n, d), jnp.float32),
    )(partials, h_t, a)


def kernel(x_h, edge_index, W, b, a):
    n, d = x_h.shape
    e = edge_index.shape[1]
    per_w = e // _NW
    chunk = 125
    nch = per_w // chunk
    assert per_w * _NW == e and nch * chunk == per_w

    h_t = _compute_ht(x_h, W, b)
    src_r = edge_index[0].reshape(_NW, nch, chunk)
    dst_r = edge_index[1].reshape(_NW, nch, chunk)
    pidx = jnp.stack([src_r, dst_r], axis=2).reshape(_NW, 2 * nch, chunk)
    partials = _sc_agg(h_t, pidx)
    return _post(partials, h_t, a)


# speedup vs baseline: 10.4931x; 1.0070x over previous
import functools

import jax
import jax.numpy as jnp
from jax import lax
from jax.experimental import pallas as pl
from jax.experimental.pallas import tpu as pltpu
from jax.experimental.pallas import tpu_sc as plsc

C = 1.0
SK = 1.0
MIN_NORM = 1e-15


def _artanh(x):
    xc = jnp.clip(x, -1.0 + 1e-7, 1.0 - 1e-7)
    return 0.5 * jnp.log((1.0 + xc) / (1.0 - xc))


def _rownorm(x):
    return jnp.maximum(jnp.sqrt(jnp.sum(x * x, axis=-1, keepdims=True)), MIN_NORM)


def _ht_body(x_ref, w_ref, b_ref, o_ref):
    x = x_ref[...]
    w = w_ref[...]
    mx = lax.dot_general(x, w, (((1,), (1,)), ((), ())),
                         preferred_element_type=jnp.float32)
    xn = _rownorm(x)
    mxn = _rownorm(mx)
    res = jnp.tanh(mxn / xn * _artanh(SK * xn)) * mx / (mxn * SK)
    zero_row = jnp.all(mx == 0, axis=-1, keepdims=True)
    h = jnp.where(zero_row, jnp.zeros_like(res), res)
    bvec = b_ref[...]
    bn = _rownorm(bvec)
    b_h = jnp.tanh(SK * bn) * bvec / (SK * bn)
    k = -C
    x2 = jnp.sum(h * h, axis=-1, keepdims=True)
    y2 = jnp.sum(b_h * b_h, axis=-1, keepdims=True)
    xy = jnp.sum(h * b_h, axis=-1, keepdims=True)
    num = (1.0 - 2.0 * k * xy - k * y2) * h + (1.0 + k * x2) * b_h
    den = 1.0 - 2.0 * k * xy + (k * k) * x2 * y2
    h = num / jnp.maximum(den, MIN_NORM)
    hn = _rownorm(h)
    o_ref[...] = _artanh(SK * hn) * h / (SK * hn)


def _compute_ht(x_h, W, b):
    n, d = x_h.shape
    blk = 2000
    grid = n // blk
    return pl.pallas_call(
        _ht_body,
        grid=(grid,),
        in_specs=[
            pl.BlockSpec((blk, d), lambda i: (i, 0)),
            pl.BlockSpec((d, d), lambda i: (0, 0)),
            pl.BlockSpec((1, d), lambda i: (0, 0)),
        ],
        out_specs=pl.BlockSpec((blk, d), lambda i: (i, 0)),
        out_shape=jax.ShapeDtypeStruct((n, d), jnp.float32),
    )(x_h, W, b.reshape(1, d))


_NC = 2
_NS = 16
_NW = _NC * _NS


def _sc_agg(h_t, pidx):
    n, d = h_t.shape
    nch, chunk = pidx.shape[1] // 2, pidx.shape[2]
    nq = nch // 4
    assert nch % 8 == 0 and nq >= 3
    rpt = (n // _NS) & ~7
    tail0 = _NS * rpt
    tail = n - tail0

    mesh = plsc.VectorSubcoreMesh(core_axis_name="c", subcore_axis_name="s")

    @functools.partial(
        pl.kernel,
        mesh=mesh,
        out_type=jax.ShapeDtypeStruct((_NC, n, d), jnp.float32),
        scratch_types=[
            pltpu.VMEM((2, 8, chunk), jnp.int32),
            pltpu.VMEM((2, chunk, d), jnp.float32),
            pltpu.VMEM_SHARED((n, d), jnp.float32),
            pltpu.SemaphoreType.DMA,
            pltpu.SemaphoreType.DMA,
            pltpu.SemaphoreType.DMA,
        ],
    )
    def agg_kernel(ht_hbm, pidx_hbm, out_hbm,
                   ring_v, rows2_v, acc_sh, sem_i, sem_g0, sem_g1):
        sem_g = (sem_g0, sem_g1)
        cid = lax.axis_index("c")
        sid = lax.axis_index("s")
        wid = sid * _NC + cid
        row0 = sid * rpt

        my_idx = pidx_hbm.at[wid]

        def idx_start(q, s2):
            pltpu.async_copy(my_idx.at[pl.ds(8 * q, 8)], ring_v.at[s2], sem_i)

        def idx_wait(q, s2):
            pltpu.make_async_copy(my_idx.at[pl.ds(8 * q, 8)], ring_v.at[s2],
                                  sem_i).wait()

        def gat_start(g, s2, m, p2):
            pltpu.async_copy(ht_hbm.at[ring_v.at[s2].at[2 * m]],
                             rows2_v.at[p2], sem_g[p2])

        def gat_wait(g, s2, m, p2):
            pltpu.make_async_copy(ht_hbm.at[ring_v.at[s2].at[2 * m]],
                                  rows2_v.at[p2], sem_g[p2]).wait()

        def scat(g, s2, m, p2):
            pltpu.sync_copy(rows2_v.at[p2], acc_sh.at[ring_v.at[s2].at[2 * m + 1]],
                            add=True)

        idx_start(0, 0)
        pltpu.sync_copy(ht_hbm.at[pl.ds(row0, rpt)],
                        acc_sh.at[pl.ds(row0, rpt)])
        if tail:
            @pl.when(sid == _NS - 1)
            def _():
                pltpu.sync_copy(ht_hbm.at[pl.ds(tail0, tail)],
                                acc_sh.at[pl.ds(tail0, tail)])
        idx_wait(0, 0)
        gat_start(0, 0, 0, 0)
        idx_start(1, 1)
        gat_start(1, 0, 1, 1)
        plsc.subcore_barrier()
        gat_wait(0, 0, 0, 0)
        scat(0, 0, 0, 0)
        for m in (2, 3):
            gat_start(m, 0, m, m % 2)
            gat_wait(m - 1, 0, m - 1, (m - 1) % 2)
            scat(m - 1, 0, m - 1, (m - 1) % 2)

        def body(t, carry):
            for h in range(2):
                q = 2 * t + 1 + h
                s2 = (1 + h) % 2
                os2 = h % 2
                idx_wait(q, s2)
                gat_start(4 * q, s2, 0, 0)
                gat_wait(4 * q - 1, os2, 3, 1)
                scat(4 * q - 1, os2, 3, 1)
                idx_start(q + 1, os2)
                for m in (1, 2, 3):
                    gat_start(4 * q + m, s2, m, m % 2)
                    gat_wait(4 * q + m - 1, s2, m - 1, (m - 1) % 2)
                    scat(4 * q + m - 1, s2, m - 1, (m - 1) % 2)
            return carry

        lax.fori_loop(0, (nq - 2) // 2, body, 0)

        q = nq - 1
        idx_wait(q, 1)
        gat_start(4 * q, 1, 0, 0)
        gat_wait(4 * q - 1, 0, 3, 1)
        scat(4 * q - 1, 0, 3, 1)
        for m in (1, 2, 3):
            gat_start(4 * q + m, 1, m, m % 2)
            gat_wait(4 * q + m - 1, 1, m - 1, (m - 1) % 2)
            scat(4 * q + m - 1, 1, m - 1, (m - 1) % 2)
        gat_wait(nch - 1, 1, 3, 1)
        scat(nch - 1, 1, 3, 1)

        plsc.subcore_barrier()
        pltpu.sync_copy(acc_sh.at[pl.ds(row0, rpt)],
                        out_hbm.at[cid].at[pl.ds(row0, rpt)])
        if tail:
            @pl.when(sid == _NS - 1)
            def _():
                pltpu.sync_copy(acc_sh.at[pl.ds(tail0, tail)],
                                out_hbm.at[cid].at[pl.ds(tail0, tail)])

    return agg_kernel(h_t, pidx)


def _post_body(p_ref, ht_ref, a_ref, o_ref):
    agg = p_ref[0] + p_ref[1] - ht_ref[...]
    an = _rownorm(agg)
    h2 = jnp.tanh(SK * an) * agg / (SK * an)
    h2n = _rownorm(h2)
    h2t = _artanh(SK * h2n) * h2 / (SK * h2n)
    a = a_ref[0]
    h2t = jnp.where(h2t >= 0, h2t, a * h2t)
    tn = _rownorm(h2t)
    h2 = jnp.tanh(SK * tn) * h2t / (SK * tn)
    eps = 4e-3
    maxnorm = (1.0 - eps) / SK
    nrm = _rownorm(h2)
    o_ref[...] = jnp.where(nrm > maxnorm, h2 / nrm * maxnorm, h2)


def _post(partials, h_t, a):
    n, d = h_t.shape
    blk = 2000
    grid = n // blk
    return pl.pallas_call(
        _post_body,
        grid=(grid,),
        in_specs=[
            pl.BlockSpec((_NC, blk, d), lambda i: (0, i, 0)),
            pl.BlockSpec((blk, d), lambda i: (i, 0)),
            pl.BlockSpec(memory_space=pltpu.SMEM),
        ],
        out_specs=pl.BlockSpec((blk, d), lambda i: (i, 0)),
        out_shape=jax.ShapeDtypeStruct((n, d), jnp.float32),
    )(partials, h_t, a)


def kernel(x_h, edge_index, W, b, a):
    n, d = x_h.shape
    e = edge_index.shape[1]
    per_w = e // _NW
    chunk = 125
    nch = per_w // chunk
    assert per_w * _NW == e and nch * chunk == per_w

    h_t = _compute_ht(x_h, W, b)
    src_r = edge_index[0].reshape(_NW, nch, chunk)
    dst_r = edge_index[1].reshape(_NW, nch, chunk)
    pidx = jnp.stack([src_r, dst_r], axis=2).reshape(_NW, 2 * nch, chunk)
    partials = _sc_agg(h_t, pidx)
    return _post(partials, h_t, a)
